# baseline (speedup 1.0000x reference)
import functools
import numpy as np
import jax
import jax.numpy as jnp
from jax import lax
from jax.experimental import pallas as pl
from jax.experimental.pallas import tpu as pltpu
from jax.experimental.pallas import tpu_sc as plsc

_P = 0.5
_MAX_DROP = 8


def _drop_indices():
    rng = np.random.RandomState(1)
    if not (rng.rand() < _P):
        return np.zeros((0,), np.int32)
    num_drop = int(rng.randint(1, _MAX_DROP + 1))
    return np.sort(rng.permutation(96)[:num_drop].astype(np.int32))


_DROP = tuple(int(i) for i in _drop_indices())

_B, _C, _H, _W = 16, 96, 224, 224
_NC, _NS = 2, 16
_NW = _NC * _NS
_PPW = (_B * _C) // _NW
_ZROWS = 56


def _sc_body(in_hbm, out_hbm, buf, zbuf, gsems, ssems, zsem):
    wid = lax.axis_index("s") * _NC + lax.axis_index("c")
    bb = wid // 2
    c_base = (wid % 2) * _PPW

    zv = jnp.zeros((16,), jnp.float32)
    for i in range(_ZROWS):
        for j in range(_W // 16):
            zbuf[i, pl.ds(j * 16, 16)] = zv

    def gather(i, j):
        return pltpu.make_async_copy(
            in_hbm.at[bb, c_base + i], buf.at[j], gsems.at[j])

    def scatter(i, j):
        return pltpu.make_async_copy(
            buf.at[j], out_hbm.at[bb, c_base + i], ssems.at[j])

    gather(0, 0).start()
    gather(1, 1).start()
    gather(0, 0).wait()
    scatter(0, 0).start()

    def step(k, _):
        i = 2 * k + 1
        scatter(i - 1, 0).wait()
        gather(i + 1, 0).start()
        gather(i, 1).wait()
        scatter(i, 1).start()
        i2 = i + 1
        scatter(i2 - 1, 1).wait()
        gather(i2 + 1, 1).start()
        gather(i2, 0).wait()
        scatter(i2, 0).start()
        return 0
    lax.fori_loop(0, (_PPW - 2) // 2, step, 0)

    i_last = _PPW - 1
    scatter(i_last - 1, 0).wait()
    gather(i_last, 1).wait()
    scatter(i_last, 1).start()
    scatter(i_last, 1).wait()

    for b in range(_B):
        for d in _DROP:
            owner = 2 * b + (1 if d >= _PPW else 0)

            @pl.when(wid == owner)
            def _():
                for r0 in range(0, _H, _ZROWS):
                    cp = pltpu.make_async_copy(
                        zbuf, out_hbm.at[b, d, pl.ds(r0, _ZROWS)], zsem)
                    cp.start()
                    cp.wait()


_mesh = plsc.VectorSubcoreMesh(core_axis_name="c", subcore_axis_name="s")

_sc_kernel = functools.partial(
    pl.kernel,
    out_type=jax.ShapeDtypeStruct((_B, _C, _H, _W), jnp.float32),
    mesh=_mesh,
    scratch_types=[
        pltpu.VMEM((2, _H, _W), jnp.float32),
        pltpu.VMEM((_ZROWS, _W), jnp.float32),
        pltpu.SemaphoreType.DMA((2,)),
        pltpu.SemaphoreType.DMA((2,)),
        pltpu.SemaphoreType.DMA,
    ],
)(_sc_body)


def kernel(image):
    return _sc_kernel(image)
